# baseline (speedup 1.0000x reference)
import functools

import jax
import jax.numpy as jnp
import numpy as np
from jax import lax
from jax.experimental import pallas as pl
from jax.experimental.pallas import tpu as pltpu
from jax.experimental.pallas import tpu_sc as plsc

N = 10000
NEIGH = 16
NREL = 3
H = 64
HH = 2 * H
NP = 10240
ZROWS = NREL * NP
E = N * NEIGH
EPAD = 163840
EPT = EPAD // 16
NCH = EPT // 128
HALF = 5000
HROWS = 15104
TRASH = 15000
RPT = HROWS // 16
BLK = 1000
GRID = N // BLK


def _tc_a_body(bond_ref, ang_ref, w1b_ref, r1b_ref,
               w1a_ref, r1a_ref, b1_ref, zb_ref, za_ref, root_ref):
    fb = np.linspace(0.0, 8.0, 16)
    inv_gb2 = 1.0 / (0.5 ** 2)
    bond = bond_ref[...]
    ef = jnp.concatenate(
        [jnp.exp(-(bond - fb[s]) ** 2 * inv_gb2) for s in range(16)], axis=1)
    fa = np.linspace(-1.0, 1.0, 2)
    ang = ang_ref[...]
    af = jnp.concatenate(
        [jnp.exp(-(ang - fa[s]) ** 2) for s in range(2)], axis=1)
    for r in range(NREL):
        zb_ref[r] = jnp.dot(ef, w1b_ref[r], preferred_element_type=jnp.float32)
        za_ref[r] = jnp.dot(af, w1a_ref[r], preferred_element_type=jnp.float32)
    root_ref[...] = jnp.concatenate(
        [jnp.dot(ef, r1b_ref[...], preferred_element_type=jnp.float32),
         jnp.dot(af, r1a_ref[...], preferred_element_type=jnp.float32)],
        axis=1) + b1_ref[...]


def _combine(acc_ref, cnt_ref, root_half):
    acc = acc_ref[0].reshape(BLK, NREL, H)
    cnt = cnt_ref[0].reshape(BLK, NREL, 16)[:, :, 0:1]
    m = acc / jnp.maximum(cnt, 1.0)
    return jax.nn.relu(root_half + m[:, 0] + m[:, 1] + m[:, 2])


def _tc_b_body(accb_ref, acca_ref, cnt_ref, root_ref, w2b_ref, r2b_ref,
               w2a_ref, r2a_ref, b2_ref, zb_ref, za_ref, root2_ref):
    xb = _combine(accb_ref, cnt_ref, root_ref[:, :H])
    xa = _combine(acca_ref, cnt_ref, root_ref[:, H:])
    for r in range(NREL):
        zb_ref[r] = jnp.dot(xb, w2b_ref[r], preferred_element_type=jnp.float32)
        za_ref[r] = jnp.dot(xa, w2a_ref[r], preferred_element_type=jnp.float32)
    root2_ref[...] = jnp.concatenate(
        [jnp.dot(xb, r2b_ref[...], preferred_element_type=jnp.float32),
         jnp.dot(xa, r2a_ref[...], preferred_element_type=jnp.float32)],
        axis=1) + b2_ref[...]


def _tc_c_body(accb_ref, acca_ref, cnt_ref, root_ref, fcw_ref, fcb_ref,
               out_ref):
    xb = _combine(accb_ref, cnt_ref, root_ref[:, :H])
    xa = _combine(acca_ref, cnt_ref, root_ref[:, H:])
    out_ref[...] = (jnp.dot(xb, fcw_ref[0], preferred_element_type=jnp.float32)
                    + jnp.dot(xa, fcw_ref[1], preferred_element_type=jnp.float32)
                    + fcb_ref[...])


def _sc_index_body(spec_hbm, nbr_hbm, ones_hbm, zer_hbm, trash_hbm, zidx_hbm,
                   seg_hbm, gidx_hbm, cnt_hbm, ecnt_hbm,
                   spec_v, dst_v, seg2_v, segc_v, gidxc_v, ones_v, ecnt_v,
                   cnt_sh, sem):
    cid = lax.axis_index("c")
    sid = lax.axis_index("s")
    pltpu.sync_copy(spec_hbm, spec_v)
    pltpu.sync_copy(nbr_hbm.at[sid], dst_v)
    pltpu.sync_copy(ones_hbm, ones_v)
    pltpu.sync_copy(trash_hbm, segc_v.at[pl.ds(0, EPT)])
    pltpu.sync_copy(zidx_hbm, gidxc_v.at[pl.ds(0, EPT)])
    ebase = sid * EPT
    lower = cid * HALF

    def chunk(c, cur):
        for k in range(8):
            lid = c * 128 + k * 16 + lax.iota(jnp.int32, 16)
            ev = ebase + lid
            dst16 = dst_v[c, pl.ds(k * 16, 16)]
            src16 = lax.shift_right_logical(ev, 4)
            sd16 = plsc.load_gather(spec_v, [dst16])
            st16 = plsc.load_gather(spec_v, [src16])
            et16 = jnp.where((st16 == 0) & (sd16 == 0), 0,
                             jnp.where((st16 == 1) & (sd16 == 1), 2, 1))
            own = (ev < E) & (dst16 >= lower) & (dst16 < lower + HALF)
            seg16 = dst16 * 3 + et16 - lower * 3
            seg2_v[c, pl.ds(k * 16, 16)] = jnp.where(own, seg16, TRASH)
            plsc.store_compressed(segc_v.at[pl.ds(cur, 16)], seg16, mask=own)
            plsc.store_compressed(gidxc_v.at[pl.ds(cur, 16)],
                                  et16 * NP + src16, mask=own)
            cur = cur + plsc.all_reduce_population_count(own)[0]
        return cur

    total = lax.fori_loop(0, NCH, chunk, jnp.int32(0))
    for j in range(2):
        segc_v[pl.ds(total + j * 16, 16)] = lax.broadcast(
            jnp.int32(TRASH), (16,))
        gidxc_v[pl.ds(total + j * 16, 16)] = lax.broadcast(jnp.int32(0), (16,))
    pltpu.sync_copy(segc_v.at[pl.ds(0, EPT)], seg_hbm.at[cid, sid])
    pltpu.sync_copy(gidxc_v.at[pl.ds(0, EPT)], gidx_hbm.at[cid, sid])
    ecnt_v[...] = lax.broadcast(total, (16,))
    pltpu.sync_copy(ecnt_v, ecnt_hbm.at[cid, sid])
    pltpu.sync_copy(zer_hbm, cnt_sh.at[pl.ds(sid * RPT, RPT)])
    plsc.subcore_barrier()

    def cscat(c, _):
        pltpu.sync_copy(ones_v, cnt_sh.at[seg2_v.at[c]], add=True)
        return _

    lax.fori_loop(0, NCH, cscat, None)
    plsc.subcore_barrier()
    pltpu.sync_copy(cnt_sh.at[pl.ds(sid * RPT, RPT)],
                    cnt_hbm.at[cid, pl.ds(sid * RPT, RPT)])


def _sc_scatter_body(zb_hbm, za_hbm, seg_hbm, gidx_hbm, zer_hbm, ecnt_hbm,
                     accb_hbm, acca_hbm,
                     seg_v, gidx_v, r0, r1, r2, r3, ecnt_v, acc_sh,
                     g0, g1, g2, g3, s0, s1, s2, s3):
    rows = (r0, r1, r2, r3)
    gsem = (g0, g1, g2, g3)
    ssem = (s0, s1, s2, s3)
    cid = lax.axis_index("c")
    sid = lax.axis_index("s")
    pltpu.sync_copy(seg_hbm.at[cid, sid], seg_v)
    pltpu.sync_copy(gidx_hbm.at[cid, sid], gidx_v)
    pltpu.sync_copy(ecnt_hbm.at[cid, sid], ecnt_v)
    total = lax.reduce_max(ecnt_v[...], axes=(0,))
    nq = (total + 511) >> 9
    nchunks = nq * 4
    for z_hbm, out_hbm in ((zb_hbm, accb_hbm), (za_hbm, acca_hbm)):
        pltpu.sync_copy(zer_hbm, acc_sh.at[pl.ds(sid * RPT, RPT)])
        plsc.subcore_barrier()
        for b in range(4):
            @pl.when(b < nchunks)
            def _prime():
                pltpu.async_copy(z_hbm.at[gidx_v.at[b]], rows[b], gsem[b])

        def quad(q, _):
            c0 = q * 4
            for b in range(4):
                c = c0 + b
                pltpu.make_async_copy(z_hbm.at[gidx_v.at[c]], rows[b],
                                      gsem[b]).wait()
                pltpu.async_copy(rows[b], acc_sh.at[seg_v.at[c]],
                                 ssem[b], add=True)
            for b in range(4):
                c = c0 + b
                pltpu.make_async_copy(rows[b], acc_sh.at[seg_v.at[c]],
                                      ssem[b]).wait()

                @pl.when(c + 4 < nchunks)
                def _prefetch():
                    pltpu.async_copy(z_hbm.at[gidx_v.at[c + 4]],
                                     rows[b], gsem[b])
            return _

        lax.fori_loop(0, nq, quad, None)
        plsc.subcore_barrier()
        pltpu.sync_copy(acc_sh.at[pl.ds(sid * RPT, RPT)],
                        out_hbm.at[cid, pl.ds(sid * RPT, RPT)])
        plsc.subcore_barrier()


@functools.lru_cache(maxsize=1)
def _sc_kernels():
    mesh = plsc.VectorSubcoreMesh(core_axis_name="c", subcore_axis_name="s")
    params = pltpu.CompilerParams(needs_layout_passes=False,
                                  use_tc_tiling_on_sc=False)
    sc_index = pl.kernel(
        _sc_index_body,
        out_type=[jax.ShapeDtypeStruct((2, 16, EPT), jnp.int32),
                  jax.ShapeDtypeStruct((2, 16, EPT), jnp.int32),
                  jax.ShapeDtypeStruct((2, HROWS, 16), jnp.float32),
                  jax.ShapeDtypeStruct((2, 16, 16), jnp.int32)],
        mesh=mesh,
        scratch_types=[pltpu.VMEM((NP,), jnp.int32),
                       pltpu.VMEM((NCH, 128), jnp.int32),
                       pltpu.VMEM((NCH, 128), jnp.int32),
                       pltpu.VMEM((EPT + 128,), jnp.int32),
                       pltpu.VMEM((EPT + 128,), jnp.int32),
                       pltpu.VMEM((128, 16), jnp.float32),
                       pltpu.VMEM((16,), jnp.int32),
                       pltpu.VMEM_SHARED((HROWS, 16), jnp.float32),
                       pltpu.SemaphoreType.DMA],
        compiler_params=params)
    sc_scatter = pl.kernel(
        _sc_scatter_body,
        out_type=[jax.ShapeDtypeStruct((2, HROWS, H), jnp.float32),
                  jax.ShapeDtypeStruct((2, HROWS, H), jnp.float32)],
        mesh=mesh,
        scratch_types=[pltpu.VMEM((NCH, 128), jnp.int32),
                       pltpu.VMEM((NCH, 128), jnp.int32)]
                      + [pltpu.VMEM((128, H), jnp.float32)] * 4
                      + [pltpu.VMEM((16,), jnp.int32),
                         pltpu.VMEM_SHARED((HROWS, H), jnp.float32)]
                      + [pltpu.SemaphoreType.DMA] * 8,
        compiler_params=params)
    return sc_index, sc_scatter


def _tc_a(bond, ang2d, w1b, r1b, w1a, r1a, b1):
    return pl.pallas_call(
        _tc_a_body,
        grid=(GRID,),
        in_specs=[
            pl.BlockSpec((BLK, NEIGH), lambda i: (i, 0)),
            pl.BlockSpec((BLK, 256), lambda i: (i, 0)),
            pl.BlockSpec((NREL, 256, H), lambda i: (0, 0, 0)),
            pl.BlockSpec((256, H), lambda i: (0, 0)),
            pl.BlockSpec((NREL, 512, H), lambda i: (0, 0, 0)),
            pl.BlockSpec((512, H), lambda i: (0, 0)),
            pl.BlockSpec((1, HH), lambda i: (0, 0)),
        ],
        out_specs=[
            pl.BlockSpec((NREL, BLK, H), lambda i: (0, i, 0)),
            pl.BlockSpec((NREL, BLK, H), lambda i: (0, i, 0)),
            pl.BlockSpec((BLK, HH), lambda i: (i, 0)),
        ],
        out_shape=[
            jax.ShapeDtypeStruct((NREL, NP, H), jnp.float32),
            jax.ShapeDtypeStruct((NREL, NP, H), jnp.float32),
            jax.ShapeDtypeStruct((N, HH), jnp.float32),
        ],
    )(bond, ang2d, w1b, r1b, w1a, r1a, b1)


_ACC_SPEC = pl.BlockSpec((1, NREL * BLK, H), lambda i: (i // 5, i % 5, 0))
_CNT_SPEC = pl.BlockSpec((1, NREL * BLK, 16), lambda i: (i // 5, i % 5, 0))


def _tc_b(accb, acca, cnt, root, w2b, r2b, w2a, r2a, b2):
    return pl.pallas_call(
        _tc_b_body,
        grid=(GRID,),
        in_specs=[
            _ACC_SPEC,
            _ACC_SPEC,
            _CNT_SPEC,
            pl.BlockSpec((BLK, HH), lambda i: (i, 0)),
            pl.BlockSpec((NREL, H, H), lambda i: (0, 0, 0)),
            pl.BlockSpec((H, H), lambda i: (0, 0)),
            pl.BlockSpec((NREL, H, H), lambda i: (0, 0, 0)),
            pl.BlockSpec((H, H), lambda i: (0, 0)),
            pl.BlockSpec((1, HH), lambda i: (0, 0)),
        ],
        out_specs=[
            pl.BlockSpec((NREL, BLK, H), lambda i: (0, i, 0)),
            pl.BlockSpec((NREL, BLK, H), lambda i: (0, i, 0)),
            pl.BlockSpec((BLK, HH), lambda i: (i, 0)),
        ],
        out_shape=[
            jax.ShapeDtypeStruct((NREL, NP, H), jnp.float32),
            jax.ShapeDtypeStruct((NREL, NP, H), jnp.float32),
            jax.ShapeDtypeStruct((N, HH), jnp.float32),
        ],
    )(accb, acca, cnt, root, w2b, r2b, w2a, r2a, b2)


def _tc_c(accb, acca, cnt, root, fcw, fcb2):
    return pl.pallas_call(
        _tc_c_body,
        grid=(GRID,),
        in_specs=[
            _ACC_SPEC,
            _ACC_SPEC,
            _CNT_SPEC,
            pl.BlockSpec((BLK, HH), lambda i: (i, 0)),
            pl.BlockSpec((2, H, 2), lambda i: (0, 0, 0)),
            pl.BlockSpec((1, 2), lambda i: (0, 0)),
        ],
        out_specs=pl.BlockSpec((BLK, 2), lambda i: (i, 0)),
        out_shape=jax.ShapeDtypeStruct((N, 2), jnp.float32),
    )(accb, acca, cnt, root, fcw, fcb2)


@jax.jit
def _run(bond_fea, angle_fea, species, nbr_idx,
         W1b, R1b, b1b, W1a, R1a, b1a,
         W2b, R2b, b2b, W2a, R2a, b2a, fcW, fcb):
    f32 = jnp.float32
    w1b = W1b.reshape(NREL, 16, 16, H).transpose(0, 2, 1, 3).reshape(NREL, 256, H)
    r1b = R1b.reshape(16, 16, H).transpose(1, 0, 2).reshape(256, H)
    w1a = W1a.reshape(NREL, 256, 2, H).transpose(0, 2, 1, 3).reshape(NREL, 512, H)
    r1a = R1a.reshape(256, 2, H).transpose(1, 0, 2).reshape(512, H)
    b1 = jnp.concatenate([b1b, b1a]).reshape(1, HH)
    b2 = jnp.concatenate([b2b, b2a]).reshape(1, HH)
    ang2d = angle_fea.reshape(N, 256)
    spec = jnp.pad(species.astype(jnp.int32), (0, NP - N))
    nbr3 = jnp.pad(nbr_idx.reshape(-1).astype(jnp.int32),
                   (0, EPAD - E)).reshape(16, NCH, 128)
    ones_h = jnp.ones((128, 16), f32)
    zer16 = jnp.zeros((RPT, 16), f32)
    zer64 = jnp.zeros((RPT, H), f32)
    trash_h = jnp.full((EPT,), TRASH, jnp.int32)
    zidx_h = jnp.zeros((EPT,), jnp.int32)

    sc_index, sc_scatter = _sc_kernels()
    seg3, gidx3, cnt, ecnt = sc_index(spec, nbr3, ones_h, zer16,
                                      trash_h, zidx_h)
    seg4 = seg3.reshape(2, 16, NCH, 128)
    gidx4 = gidx3.reshape(2, 16, NCH, 128)
    z1b, z1a, root1 = _tc_a(bond_fea, ang2d, w1b, r1b, w1a, r1a, b1)
    acc1b, acc1a = sc_scatter(z1b.reshape(ZROWS, H), z1a.reshape(ZROWS, H),
                              seg4, gidx4, zer64, ecnt)
    z2b, z2a, root2 = _tc_b(acc1b, acc1a, cnt, root1, W2b, R2b, W2a, R2a, b2)
    acc2b, acc2a = sc_scatter(z2b.reshape(ZROWS, H), z2a.reshape(ZROWS, H),
                              seg4, gidx4, zer64, ecnt)
    return _tc_c(acc2b, acc2a, cnt, root2,
                 fcW.reshape(2, H, 2), fcb.reshape(1, 2))


def kernel(bond_fea, angle_fea, species, nbr_idx, crys_idx,
           W1b, R1b, b1b, W1a, R1a, b1a,
           W2b, R2b, b2b, W2a, R2a, b2a, fcW, fcb):
    del crys_idx
    return _run(bond_fea, angle_fea, species, nbr_idx,
                W1b, R1b, b1b, W1a, R1a, b1a,
                W2b, R2b, b2b, W2a, R2a, b2a, fcW, fcb)


# speedup vs baseline: 1.5107x; 1.5107x over previous
import functools

import jax
import jax.numpy as jnp
import numpy as np
from jax import lax
from jax.experimental import pallas as pl
from jax.experimental.pallas import tpu as pltpu
from jax.experimental.pallas import tpu_sc as plsc

N = 10000
NEIGH = 16
NREL = 3
H = 64
HH = 2 * H
NP = 10240
ZROWS = NREL * NP
E = N * NEIGH
EPAD = 163840
EPT = EPAD // 16
NCH = EPT // 128
HALF = 5000
PLANE = 5120
HROWS = NREL * PLANE
TRASH = PLANE - 1
RPT = HROWS // 16
BLK = 1000
GRID = N // BLK


def _mm(x, w):
    return jnp.dot(x, w, preferred_element_type=jnp.float32)


def _tc_a_body(bond_ref, ang_ref, w1b_ref, w1a_ref, b1_ref,
               zb_ref, za_ref, root_ref):
    fb = np.linspace(0.0, 8.0, 16)
    inv_gb2 = 1.0 / (0.5 ** 2)
    bond = bond_ref[...]
    ef = jnp.concatenate(
        [jnp.exp(-(bond - fb[s]) ** 2 * inv_gb2) for s in range(16)], axis=1)
    fa = np.linspace(-1.0, 1.0, 2)
    ang = ang_ref[...]
    af = jnp.concatenate(
        [jnp.exp(-(ang - fa[s]) ** 2) for s in range(2)], axis=1)
    zb_all = _mm(ef, w1b_ref[...])
    za_all = _mm(af, w1a_ref[...])
    for r in range(NREL):
        zb_ref[r] = zb_all[:, r * H:(r + 1) * H]
        za_ref[r] = za_all[:, r * H:(r + 1) * H]
    root_ref[...] = jnp.concatenate(
        [zb_all[:, NREL * H:], za_all[:, NREL * H:]], axis=1) + b1_ref[...]


def _combine(acc_ref, cnt_ref, root_half):
    acc = acc_ref[0]
    cnt = cnt_ref[0][:, :, 0:1]
    inv = 1.0 / jnp.maximum(cnt, 1.0)
    m = acc * inv
    return jax.nn.relu(root_half + m[0] + m[1] + m[2])


def _tc_b_body(accb_ref, acca_ref, cnt_ref, root_ref, w2b_ref,
               w2a_ref, b2_ref, zb_ref, za_ref, root2_ref):
    xb = _combine(accb_ref, cnt_ref, root_ref[:, :H])
    xa = _combine(acca_ref, cnt_ref, root_ref[:, H:])
    zb_all = _mm(xb, w2b_ref[...])
    za_all = _mm(xa, w2a_ref[...])
    for r in range(NREL):
        zb_ref[r] = zb_all[:, r * H:(r + 1) * H]
        za_ref[r] = za_all[:, r * H:(r + 1) * H]
    root2_ref[...] = jnp.concatenate(
        [zb_all[:, NREL * H:], za_all[:, NREL * H:]], axis=1) + b2_ref[...]


def _tc_c_body(accb_ref, acca_ref, cnt_ref, root_ref, fcw_ref, fcb_ref,
               out_ref):
    xb = _combine(accb_ref, cnt_ref, root_ref[:, :H])
    xa = _combine(acca_ref, cnt_ref, root_ref[:, H:])
    x = jnp.concatenate([xb, xa], axis=1)
    out_ref[...] = _mm(x, fcw_ref[...]) + fcb_ref[...]


def _sc_index_body(spec_hbm, nbr_hbm, ones_hbm, zer_hbm, trash_hbm, zidx_hbm,
                   seg_hbm, gidx_hbm, cnt_hbm, ecnt_hbm,
                   spec_v, dst_v, seg2_v, segc_v, gidxc_v, ones_v, ecnt_v,
                   cnt_sh, sem):
    cid = lax.axis_index("c")
    sid = lax.axis_index("s")
    pltpu.sync_copy(spec_hbm, spec_v)
    pltpu.sync_copy(nbr_hbm.at[sid], dst_v)
    pltpu.sync_copy(ones_hbm, ones_v)
    pltpu.sync_copy(trash_hbm, segc_v.at[pl.ds(0, EPT)])
    pltpu.sync_copy(zidx_hbm, gidxc_v.at[pl.ds(0, EPT)])
    ebase = sid * EPT
    lower = cid * HALF

    def chunk(c, cur):
        for k in range(8):
            lid = c * 128 + k * 16 + lax.iota(jnp.int32, 16)
            ev = ebase + lid
            dst16 = dst_v[c, pl.ds(k * 16, 16)]
            src16 = lax.shift_right_logical(ev, 4)
            sd16 = plsc.load_gather(spec_v, [dst16])
            st16 = plsc.load_gather(spec_v, [src16])
            et16 = jnp.where((st16 == 0) & (sd16 == 0), 0,
                             jnp.where((st16 == 1) & (sd16 == 1), 2, 1))
            own = (ev < E) & (dst16 >= lower) & (dst16 < lower + HALF)
            seg16 = et16 * PLANE + dst16 - lower
            seg2_v[c, pl.ds(k * 16, 16)] = jnp.where(own, seg16, TRASH)
            plsc.store_compressed(segc_v.at[pl.ds(cur, 16)], seg16, mask=own)
            plsc.store_compressed(gidxc_v.at[pl.ds(cur, 16)],
                                  et16 * NP + src16, mask=own)
            cur = cur + plsc.all_reduce_population_count(own)[0]
        return cur

    total = lax.fori_loop(0, NCH, chunk, jnp.int32(0))
    for j in range(2):
        segc_v[pl.ds(total + j * 16, 16)] = lax.broadcast(
            jnp.int32(TRASH), (16,))
        gidxc_v[pl.ds(total + j * 16, 16)] = lax.broadcast(jnp.int32(0), (16,))
    pltpu.sync_copy(segc_v.at[pl.ds(0, EPT)], seg_hbm.at[cid, sid])
    pltpu.sync_copy(gidxc_v.at[pl.ds(0, EPT)], gidx_hbm.at[cid, sid])
    ecnt_v[...] = lax.broadcast(total, (16,))
    pltpu.sync_copy(ecnt_v, ecnt_hbm.at[cid, sid])
    pltpu.sync_copy(zer_hbm, cnt_sh.at[pl.ds(sid * RPT, RPT)])
    plsc.subcore_barrier()

    def cscat(c, _):
        pltpu.sync_copy(ones_v, cnt_sh.at[seg2_v.at[c]], add=True)
        return _

    lax.fori_loop(0, NCH, cscat, None)
    plsc.subcore_barrier()
    pltpu.sync_copy(cnt_sh.at[pl.ds(sid * RPT, RPT)],
                    cnt_hbm.at[cid, pl.ds(sid * RPT, RPT)])


def _sc_scatter_body(zb_hbm, za_hbm, seg_hbm, gidx_hbm, zer_hbm, ecnt_hbm,
                     accb_hbm, acca_hbm,
                     seg_v, gidx_v, rows0, rows1, ecnt_v, acc_sh, sem0, sem1):
    cid = lax.axis_index("c")
    sid = lax.axis_index("s")
    pltpu.sync_copy(seg_hbm.at[cid, sid], seg_v)
    pltpu.sync_copy(gidx_hbm.at[cid, sid], gidx_v)
    pltpu.sync_copy(ecnt_hbm.at[cid, sid], ecnt_v)
    total = lax.reduce_max(ecnt_v[...], axes=(0,))
    npairs = (total + 255) >> 8
    for z_hbm, out_hbm in ((zb_hbm, accb_hbm), (za_hbm, acca_hbm)):
        pltpu.sync_copy(zer_hbm, acc_sh.at[pl.ds(sid * RPT, RPT)])
        plsc.subcore_barrier()

        @pl.when(npairs > 0)
        def _prime():
            pltpu.async_copy(z_hbm.at[gidx_v.at[0]], rows0, sem0)

        def pair(p, _):
            c0 = 2 * p
            pltpu.async_copy(z_hbm.at[gidx_v.at[c0 + 1]], rows1, sem1)
            pltpu.make_async_copy(z_hbm.at[gidx_v.at[c0]], rows0, sem0).wait()
            pltpu.sync_copy(rows0, acc_sh.at[seg_v.at[c0]], add=True)

            @pl.when(p < npairs - 1)
            def _prefetch():
                pltpu.async_copy(z_hbm.at[gidx_v.at[c0 + 2]], rows0, sem0)

            pltpu.make_async_copy(z_hbm.at[gidx_v.at[c0 + 1]], rows1, sem1).wait()
            pltpu.sync_copy(rows1, acc_sh.at[seg_v.at[c0 + 1]], add=True)
            return _

        lax.fori_loop(0, npairs, pair, None)
        plsc.subcore_barrier()
        pltpu.sync_copy(acc_sh.at[pl.ds(sid * RPT, RPT)],
                        out_hbm.at[cid, pl.ds(sid * RPT, RPT)])
        plsc.subcore_barrier()


@functools.lru_cache(maxsize=1)
def _sc_kernels():
    mesh = plsc.VectorSubcoreMesh(core_axis_name="c", subcore_axis_name="s")
    params = pltpu.CompilerParams(needs_layout_passes=False,
                                  use_tc_tiling_on_sc=False)
    sc_index = pl.kernel(
        _sc_index_body,
        out_type=[jax.ShapeDtypeStruct((2, 16, EPT), jnp.int32),
                  jax.ShapeDtypeStruct((2, 16, EPT), jnp.int32),
                  jax.ShapeDtypeStruct((2, HROWS, 16), jnp.float32),
                  jax.ShapeDtypeStruct((2, 16, 16), jnp.int32)],
        mesh=mesh,
        scratch_types=[pltpu.VMEM((NP,), jnp.int32),
                       pltpu.VMEM((NCH, 128), jnp.int32),
                       pltpu.VMEM((NCH, 128), jnp.int32),
                       pltpu.VMEM((EPT + 128,), jnp.int32),
                       pltpu.VMEM((EPT + 128,), jnp.int32),
                       pltpu.VMEM((128, 16), jnp.float32),
                       pltpu.VMEM((16,), jnp.int32),
                       pltpu.VMEM_SHARED((HROWS, 16), jnp.float32),
                       pltpu.SemaphoreType.DMA],
        compiler_params=params)
    sc_scatter = pl.kernel(
        _sc_scatter_body,
        out_type=[jax.ShapeDtypeStruct((2, HROWS, H), jnp.float32),
                  jax.ShapeDtypeStruct((2, HROWS, H), jnp.float32)],
        mesh=mesh,
        scratch_types=[pltpu.VMEM((NCH, 128), jnp.int32),
                       pltpu.VMEM((NCH, 128), jnp.int32),
                       pltpu.VMEM((128, H), jnp.float32),
                       pltpu.VMEM((128, H), jnp.float32),
                       pltpu.VMEM((16,), jnp.int32),
                       pltpu.VMEM_SHARED((HROWS, H), jnp.float32),
                       pltpu.SemaphoreType.DMA,
                       pltpu.SemaphoreType.DMA],
        compiler_params=params)
    return sc_index, sc_scatter


def _tc_a(bond, ang2d, w1b, w1a, b1):
    return pl.pallas_call(
        _tc_a_body,
        grid=(GRID,),
        in_specs=[
            pl.BlockSpec((BLK, NEIGH), lambda i: (i, 0)),
            pl.BlockSpec((BLK, 256), lambda i: (i, 0)),
            pl.BlockSpec((256, 4 * H), lambda i: (0, 0)),
            pl.BlockSpec((512, 4 * H), lambda i: (0, 0)),
            pl.BlockSpec((1, HH), lambda i: (0, 0)),
        ],
        out_specs=[
            pl.BlockSpec((NREL, BLK, H), lambda i: (0, i, 0)),
            pl.BlockSpec((NREL, BLK, H), lambda i: (0, i, 0)),
            pl.BlockSpec((BLK, HH), lambda i: (i, 0)),
        ],
        out_shape=[
            jax.ShapeDtypeStruct((NREL, NP, H), jnp.float32),
            jax.ShapeDtypeStruct((NREL, NP, H), jnp.float32),
            jax.ShapeDtypeStruct((N, HH), jnp.float32),
        ],
    )(bond, ang2d, w1b, w1a, b1)


_ACC_SPEC = pl.BlockSpec((1, NREL, BLK, H), lambda i: (i // 5, 0, i % 5, 0))
_CNT_SPEC = pl.BlockSpec((1, NREL, BLK, 16), lambda i: (i // 5, 0, i % 5, 0))


def _tc_b(accb, acca, cnt, root, w2b, w2a, b2):
    return pl.pallas_call(
        _tc_b_body,
        grid=(GRID,),
        in_specs=[
            _ACC_SPEC,
            _ACC_SPEC,
            _CNT_SPEC,
            pl.BlockSpec((BLK, HH), lambda i: (i, 0)),
            pl.BlockSpec((H, 4 * H), lambda i: (0, 0)),
            pl.BlockSpec((H, 4 * H), lambda i: (0, 0)),
            pl.BlockSpec((1, HH), lambda i: (0, 0)),
        ],
        out_specs=[
            pl.BlockSpec((NREL, BLK, H), lambda i: (0, i, 0)),
            pl.BlockSpec((NREL, BLK, H), lambda i: (0, i, 0)),
            pl.BlockSpec((BLK, HH), lambda i: (i, 0)),
        ],
        out_shape=[
            jax.ShapeDtypeStruct((NREL, NP, H), jnp.float32),
            jax.ShapeDtypeStruct((NREL, NP, H), jnp.float32),
            jax.ShapeDtypeStruct((N, HH), jnp.float32),
        ],
    )(accb, acca, cnt, root, w2b, w2a, b2)


def _tc_c(accb, acca, cnt, root, fcw, fcb2):
    return pl.pallas_call(
        _tc_c_body,
        grid=(GRID,),
        in_specs=[
            _ACC_SPEC,
            _ACC_SPEC,
            _CNT_SPEC,
            pl.BlockSpec((BLK, HH), lambda i: (i, 0)),
            pl.BlockSpec((HH, 2), lambda i: (0, 0)),
            pl.BlockSpec((1, 2), lambda i: (0, 0)),
        ],
        out_specs=pl.BlockSpec((BLK, 2), lambda i: (i, 0)),
        out_shape=jax.ShapeDtypeStruct((N, 2), jnp.float32),
    )(accb, acca, cnt, root, fcw, fcb2)


@jax.jit
def _run(bond_fea, angle_fea, species, nbr_idx,
         W1b, R1b, b1b, W1a, R1a, b1a,
         W2b, R2b, b2b, W2a, R2a, b2a, fcW, fcb):
    f32 = jnp.float32
    w1b = W1b.reshape(NREL, 16, 16, H).transpose(0, 2, 1, 3).reshape(NREL, 256, H)
    r1b = R1b.reshape(16, 16, H).transpose(1, 0, 2).reshape(256, H)
    w1a = W1a.reshape(NREL, 256, 2, H).transpose(0, 2, 1, 3).reshape(NREL, 512, H)
    r1a = R1a.reshape(256, 2, H).transpose(1, 0, 2).reshape(512, H)
    w1bcat = jnp.concatenate([w1b[0], w1b[1], w1b[2], r1b], axis=1)
    w1acat = jnp.concatenate([w1a[0], w1a[1], w1a[2], r1a], axis=1)
    w2bcat = jnp.concatenate([W2b[0], W2b[1], W2b[2], R2b], axis=1)
    w2acat = jnp.concatenate([W2a[0], W2a[1], W2a[2], R2a], axis=1)
    b1 = jnp.concatenate([b1b, b1a]).reshape(1, HH)
    b2 = jnp.concatenate([b2b, b2a]).reshape(1, HH)
    ang2d = angle_fea.reshape(N, 256)
    spec = jnp.pad(species.astype(jnp.int32), (0, NP - N))
    nbr3 = jnp.pad(nbr_idx.reshape(-1).astype(jnp.int32),
                   (0, EPAD - E)).reshape(16, NCH, 128)
    ones_h = jnp.ones((128, 16), f32)
    zer16 = jnp.zeros((RPT, 16), f32)
    zer64 = jnp.zeros((RPT, H), f32)
    trash_h = jnp.full((EPT,), TRASH, jnp.int32)
    zidx_h = jnp.zeros((EPT,), jnp.int32)

    sc_index, sc_scatter = _sc_kernels()
    seg3, gidx3, cnt, ecnt = sc_index(spec, nbr3, ones_h, zer16,
                                      trash_h, zidx_h)
    seg4 = seg3.reshape(2, 16, NCH, 128)
    gidx4 = gidx3.reshape(2, 16, NCH, 128)
    z1b, z1a, root1 = _tc_a(bond_fea, ang2d, w1bcat, w1acat, b1)
    acc1b, acc1a = sc_scatter(z1b.reshape(ZROWS, H), z1a.reshape(ZROWS, H),
                              seg4, gidx4, zer64, ecnt)
    z2b, z2a, root2 = _tc_b(acc1b.reshape(2, NREL, PLANE, H),
                            acc1a.reshape(2, NREL, PLANE, H),
                            cnt.reshape(2, NREL, PLANE, 16),
                            root1, w2bcat, w2acat, b2)
    acc2b, acc2a = sc_scatter(z2b.reshape(ZROWS, H), z2a.reshape(ZROWS, H),
                              seg4, gidx4, zer64, ecnt)
    return _tc_c(acc2b.reshape(2, NREL, PLANE, H),
                 acc2a.reshape(2, NREL, PLANE, H),
                 cnt.reshape(2, NREL, PLANE, 16),
                 root2, fcW, fcb.reshape(1, 2))


def kernel(bond_fea, angle_fea, species, nbr_idx, crys_idx,
           W1b, R1b, b1b, W1a, R1a, b1a,
           W2b, R2b, b2b, W2a, R2a, b2a, fcW, fcb):
    del crys_idx
    return _run(bond_fea, angle_fea, species, nbr_idx,
                W1b, R1b, b1b, W1a, R1a, b1a,
                W2b, R2b, b2b, W2a, R2a, b2a, fcW, fcb)


# speedup vs baseline: 1.5503x; 1.0262x over previous
import functools

import jax
import jax.numpy as jnp
import numpy as np
from jax import lax
from jax.experimental import pallas as pl
from jax.experimental.pallas import tpu as pltpu
from jax.experimental.pallas import tpu_sc as plsc

N = 10000
NEIGH = 16
NREL = 3
H = 64
HH = 2 * H
NP = 10240
ZROWS = NREL * NP
E = N * NEIGH
EPAD = 163840
EPT = EPAD // 16
NCH = EPT // 128
HALF = 5000
PLANE = 5120
HROWS = NREL * PLANE
TRASH = PLANE - 1
RPT = HROWS // 16
BLK = 1000
GRID = N // BLK


def _mm(x, w):
    return jnp.dot(x, w, preferred_element_type=jnp.float32)


def _tc_a_body(bond_ref, ang_ref, w1b_ref, w1a_ref, b1_ref,
               zb_ref, za_ref, root_ref):
    fb = np.linspace(0.0, 8.0, 16)
    inv_gb2 = 1.0 / (0.5 ** 2)
    bond = bond_ref[...]
    ef = jnp.concatenate(
        [jnp.exp(-(bond - fb[s]) ** 2 * inv_gb2) for s in range(16)], axis=1)
    fa = np.linspace(-1.0, 1.0, 2)
    ang = ang_ref[...]
    af = jnp.concatenate(
        [jnp.exp(-(ang - fa[s]) ** 2) for s in range(2)], axis=1)
    zb_all = _mm(ef, w1b_ref[...])
    za_all = _mm(af, w1a_ref[...])
    for r in range(NREL):
        zb_ref[r] = zb_all[:, r * H:(r + 1) * H]
        za_ref[r] = za_all[:, r * H:(r + 1) * H]
    root_ref[...] = jnp.concatenate(
        [zb_all[:, NREL * H:], za_all[:, NREL * H:]], axis=1) + b1_ref[...]


def _combine(acc_ref, cnt_ref, root_half):
    acc = acc_ref[0]
    cnt = cnt_ref[0][:, :, 0:1]
    inv = 1.0 / jnp.maximum(cnt, 1.0)
    m = acc * inv
    return jax.nn.relu(root_half + m[0] + m[1] + m[2])


def _tc_b_body(accb_ref, acca_ref, cnt_ref, root_ref, w2b_ref,
               w2a_ref, b2_ref, zb_ref, za_ref, root2_ref):
    xb = _combine(accb_ref, cnt_ref, root_ref[:, :H])
    xa = _combine(acca_ref, cnt_ref, root_ref[:, H:])
    zb_all = _mm(xb, w2b_ref[...])
    za_all = _mm(xa, w2a_ref[...])
    for r in range(NREL):
        zb_ref[r] = zb_all[:, r * H:(r + 1) * H]
        za_ref[r] = za_all[:, r * H:(r + 1) * H]
    root2_ref[...] = jnp.concatenate(
        [zb_all[:, NREL * H:], za_all[:, NREL * H:]], axis=1) + b2_ref[...]


def _tc_c_body(accb_ref, acca_ref, cnt_ref, root_ref, fcw_ref, fcb_ref,
               out_ref):
    xb = _combine(accb_ref, cnt_ref, root_ref[:, :H])
    xa = _combine(acca_ref, cnt_ref, root_ref[:, H:])
    x = jnp.concatenate([xb, xa], axis=1)
    out_ref[...] = _mm(x, fcw_ref[...]) + fcb_ref[...]


def _sc_index_body(spec_hbm, nbr_hbm, trash_hbm, zidx_hbm,
                   seg_hbm, gidx_hbm, ecnt_hbm,
                   spec_v, dst_v, segc_v, gidxc_v, ecnt_v, sem):
    cid = lax.axis_index("c")
    sid = lax.axis_index("s")
    pltpu.sync_copy(spec_hbm, spec_v)
    pltpu.sync_copy(nbr_hbm.at[sid], dst_v)
    pltpu.sync_copy(trash_hbm, segc_v.at[pl.ds(0, EPT)])
    pltpu.sync_copy(zidx_hbm, gidxc_v.at[pl.ds(0, EPT)])
    ebase = sid * EPT
    lower = cid * HALF

    def chunk(c, cur):
        for k in range(8):
            lid = c * 128 + k * 16 + lax.iota(jnp.int32, 16)
            ev = ebase + lid
            dst16 = dst_v[c, pl.ds(k * 16, 16)]
            src16 = lax.shift_right_logical(ev, 4)
            sd16 = plsc.load_gather(spec_v, [dst16])
            st16 = plsc.load_gather(spec_v, [src16])
            et16 = jnp.where((st16 == 0) & (sd16 == 0), 0,
                             jnp.where((st16 == 1) & (sd16 == 1), 2, 1))
            own = (ev < E) & (dst16 >= lower) & (dst16 < lower + HALF)
            seg16 = et16 * PLANE + dst16 - lower
            plsc.store_compressed(segc_v.at[pl.ds(cur, 16)], seg16, mask=own)
            plsc.store_compressed(gidxc_v.at[pl.ds(cur, 16)],
                                  et16 * NP + src16, mask=own)
            cur = cur + plsc.all_reduce_population_count(own)[0]
        return cur

    total = lax.fori_loop(0, NCH, chunk, jnp.int32(0))
    for j in range(2):
        segc_v[pl.ds(total + j * 16, 16)] = lax.broadcast(
            jnp.int32(TRASH), (16,))
        gidxc_v[pl.ds(total + j * 16, 16)] = lax.broadcast(jnp.int32(0), (16,))
    pltpu.sync_copy(segc_v.at[pl.ds(0, EPT)], seg_hbm.at[cid, sid])
    pltpu.sync_copy(gidxc_v.at[pl.ds(0, EPT)], gidx_hbm.at[cid, sid])
    ecnt_v[...] = lax.broadcast(total, (16,))
    pltpu.sync_copy(ecnt_v, ecnt_hbm.at[cid, sid])


def _scatter_common(zb_hbm, za_hbm, accb_hbm, acca_hbm, zer_hbm,
                    seg_v, gidx_v, rows0, rows1, acc_sh, sem0, sem1,
                    sid, npairs):
    for z_hbm, out_hbm in ((zb_hbm, accb_hbm), (za_hbm, acca_hbm)):
        pltpu.sync_copy(zer_hbm, acc_sh.at[pl.ds(sid * RPT, RPT)])
        plsc.subcore_barrier()

        @pl.when(npairs > 0)
        def _prime():
            pltpu.async_copy(z_hbm.at[gidx_v.at[0]], rows0, sem0)

        def pair(p, _):
            c0 = 2 * p
            pltpu.async_copy(z_hbm.at[gidx_v.at[c0 + 1]], rows1, sem1)
            pltpu.make_async_copy(z_hbm.at[gidx_v.at[c0]], rows0, sem0).wait()
            pltpu.sync_copy(rows0, acc_sh.at[seg_v.at[c0]], add=True)

            @pl.when(p < npairs - 1)
            def _prefetch():
                pltpu.async_copy(z_hbm.at[gidx_v.at[c0 + 2]], rows0, sem0)

            pltpu.make_async_copy(z_hbm.at[gidx_v.at[c0 + 1]], rows1, sem1).wait()
            pltpu.sync_copy(rows1, acc_sh.at[seg_v.at[c0 + 1]], add=True)
            return _

        lax.fori_loop(0, npairs, pair, None)
        plsc.subcore_barrier()
        cid = lax.axis_index("c")
        pltpu.sync_copy(acc_sh.at[pl.ds(sid * RPT, RPT)],
                        out_hbm.at[cid, pl.ds(sid * RPT, RPT)])
        plsc.subcore_barrier()


def _sc_scatter_body(zb_hbm, za_hbm, seg_hbm, gidx_hbm, zer_hbm, ecnt_hbm,
                     accb_hbm, acca_hbm,
                     seg_v, gidx_v, rows0, rows1, ecnt_v, acc_sh, sem0, sem1):
    cid = lax.axis_index("c")
    sid = lax.axis_index("s")
    pltpu.sync_copy(seg_hbm.at[cid, sid], seg_v)
    pltpu.sync_copy(gidx_hbm.at[cid, sid], gidx_v)
    pltpu.sync_copy(ecnt_hbm.at[cid, sid], ecnt_v)
    total = lax.reduce_max(ecnt_v[...], axes=(0,))
    npairs = (total + 255) >> 8
    _scatter_common(zb_hbm, za_hbm, accb_hbm, acca_hbm, zer_hbm,
                    seg_v, gidx_v, rows0, rows1, acc_sh, sem0, sem1,
                    sid, npairs)


def _sc_scatter_cnt_body(zb_hbm, za_hbm, seg_hbm, gidx_hbm, zer_hbm, ecnt_hbm,
                         ones_hbm, zer16_hbm,
                         accb_hbm, acca_hbm, cnt_hbm,
                         seg_v, gidx_v, rows0, rows1, ecnt_v, ones_v,
                         acc_sh, cnt_sh, sem0, sem1):
    cid = lax.axis_index("c")
    sid = lax.axis_index("s")
    pltpu.sync_copy(seg_hbm.at[cid, sid], seg_v)
    pltpu.sync_copy(gidx_hbm.at[cid, sid], gidx_v)
    pltpu.sync_copy(ecnt_hbm.at[cid, sid], ecnt_v)
    pltpu.sync_copy(ones_hbm, ones_v)
    total = lax.reduce_max(ecnt_v[...], axes=(0,))
    npairs = (total + 255) >> 8
    pltpu.sync_copy(zer16_hbm, cnt_sh.at[pl.ds(sid * RPT, RPT)])
    plsc.subcore_barrier()

    def cscat(c, _):
        pltpu.sync_copy(ones_v, cnt_sh.at[seg_v.at[c]], add=True)
        return _

    lax.fori_loop(0, 2 * npairs, cscat, None)
    plsc.subcore_barrier()
    pltpu.sync_copy(cnt_sh.at[pl.ds(sid * RPT, RPT)],
                    cnt_hbm.at[cid, pl.ds(sid * RPT, RPT)])
    _scatter_common(zb_hbm, za_hbm, accb_hbm, acca_hbm, zer_hbm,
                    seg_v, gidx_v, rows0, rows1, acc_sh, sem0, sem1,
                    sid, npairs)


@functools.lru_cache(maxsize=1)
def _sc_kernels():
    mesh = plsc.VectorSubcoreMesh(core_axis_name="c", subcore_axis_name="s")
    params = pltpu.CompilerParams(needs_layout_passes=False,
                                  use_tc_tiling_on_sc=False)
    sc_index = pl.kernel(
        _sc_index_body,
        out_type=[jax.ShapeDtypeStruct((2, 16, EPT), jnp.int32),
                  jax.ShapeDtypeStruct((2, 16, EPT), jnp.int32),
                  jax.ShapeDtypeStruct((2, 16, 16), jnp.int32)],
        mesh=mesh,
        scratch_types=[pltpu.VMEM((NP,), jnp.int32),
                       pltpu.VMEM((NCH, 128), jnp.int32),
                       pltpu.VMEM((EPT + 128,), jnp.int32),
                       pltpu.VMEM((EPT + 128,), jnp.int32),
                       pltpu.VMEM((16,), jnp.int32),
                       pltpu.SemaphoreType.DMA],
        compiler_params=params)
    sc_scatter = pl.kernel(
        _sc_scatter_body,
        out_type=[jax.ShapeDtypeStruct((2, HROWS, H), jnp.float32),
                  jax.ShapeDtypeStruct((2, HROWS, H), jnp.float32)],
        mesh=mesh,
        scratch_types=[pltpu.VMEM((NCH, 128), jnp.int32),
                       pltpu.VMEM((NCH, 128), jnp.int32),
                       pltpu.VMEM((128, H), jnp.float32),
                       pltpu.VMEM((128, H), jnp.float32),
                       pltpu.VMEM((16,), jnp.int32),
                       pltpu.VMEM_SHARED((HROWS, H), jnp.float32),
                       pltpu.SemaphoreType.DMA,
                       pltpu.SemaphoreType.DMA],
        compiler_params=params)
    sc_scatter_cnt = pl.kernel(
        _sc_scatter_cnt_body,
        out_type=[jax.ShapeDtypeStruct((2, HROWS, H), jnp.float32),
                  jax.ShapeDtypeStruct((2, HROWS, H), jnp.float32),
                  jax.ShapeDtypeStruct((2, HROWS, 16), jnp.float32)],
        mesh=mesh,
        scratch_types=[pltpu.VMEM((NCH, 128), jnp.int32),
                       pltpu.VMEM((NCH, 128), jnp.int32),
                       pltpu.VMEM((128, H), jnp.float32),
                       pltpu.VMEM((128, H), jnp.float32),
                       pltpu.VMEM((16,), jnp.int32),
                       pltpu.VMEM((128, 16), jnp.float32),
                       pltpu.VMEM_SHARED((HROWS, H), jnp.float32),
                       pltpu.VMEM_SHARED((HROWS, 16), jnp.float32),
                       pltpu.SemaphoreType.DMA,
                       pltpu.SemaphoreType.DMA],
        compiler_params=params)
    return sc_index, sc_scatter, sc_scatter_cnt


def _tc_a(bond, ang2d, w1b, w1a, b1):
    return pl.pallas_call(
        _tc_a_body,
        grid=(GRID,),
        in_specs=[
            pl.BlockSpec((BLK, NEIGH), lambda i: (i, 0)),
            pl.BlockSpec((BLK, 256), lambda i: (i, 0)),
            pl.BlockSpec((256, 4 * H), lambda i: (0, 0)),
            pl.BlockSpec((512, 4 * H), lambda i: (0, 0)),
            pl.BlockSpec((1, HH), lambda i: (0, 0)),
        ],
        out_specs=[
            pl.BlockSpec((NREL, BLK, H), lambda i: (0, i, 0)),
            pl.BlockSpec((NREL, BLK, H), lambda i: (0, i, 0)),
            pl.BlockSpec((BLK, HH), lambda i: (i, 0)),
        ],
        out_shape=[
            jax.ShapeDtypeStruct((NREL, NP, H), jnp.float32),
            jax.ShapeDtypeStruct((NREL, NP, H), jnp.float32),
            jax.ShapeDtypeStruct((N, HH), jnp.float32),
        ],
    )(bond, ang2d, w1b, w1a, b1)


_ACC_SPEC = pl.BlockSpec((1, NREL, BLK, H), lambda i: (i // 5, 0, i % 5, 0))
_CNT_SPEC = pl.BlockSpec((1, NREL, BLK, 16), lambda i: (i // 5, 0, i % 5, 0))


def _tc_b(accb, acca, cnt, root, w2b, w2a, b2):
    return pl.pallas_call(
        _tc_b_body,
        grid=(GRID,),
        in_specs=[
            _ACC_SPEC,
            _ACC_SPEC,
            _CNT_SPEC,
            pl.BlockSpec((BLK, HH), lambda i: (i, 0)),
            pl.BlockSpec((H, 4 * H), lambda i: (0, 0)),
            pl.BlockSpec((H, 4 * H), lambda i: (0, 0)),
            pl.BlockSpec((1, HH), lambda i: (0, 0)),
        ],
        out_specs=[
            pl.BlockSpec((NREL, BLK, H), lambda i: (0, i, 0)),
            pl.BlockSpec((NREL, BLK, H), lambda i: (0, i, 0)),
            pl.BlockSpec((BLK, HH), lambda i: (i, 0)),
        ],
        out_shape=[
            jax.ShapeDtypeStruct((NREL, NP, H), jnp.float32),
            jax.ShapeDtypeStruct((NREL, NP, H), jnp.float32),
            jax.ShapeDtypeStruct((N, HH), jnp.float32),
        ],
    )(accb, acca, cnt, root, w2b, w2a, b2)


def _tc_c(accb, acca, cnt, root, fcw, fcb2):
    return pl.pallas_call(
        _tc_c_body,
        grid=(GRID,),
        in_specs=[
            _ACC_SPEC,
            _ACC_SPEC,
            _CNT_SPEC,
            pl.BlockSpec((BLK, HH), lambda i: (i, 0)),
            pl.BlockSpec((HH, 2), lambda i: (0, 0)),
            pl.BlockSpec((1, 2), lambda i: (0, 0)),
        ],
        out_specs=pl.BlockSpec((BLK, 2), lambda i: (i, 0)),
        out_shape=jax.ShapeDtypeStruct((N, 2), jnp.float32),
    )(accb, acca, cnt, root, fcw, fcb2)


@jax.jit
def _run(bond_fea, angle_fea, species, nbr_idx,
         W1b, R1b, b1b, W1a, R1a, b1a,
         W2b, R2b, b2b, W2a, R2a, b2a, fcW, fcb):
    f32 = jnp.float32
    w1b = W1b.reshape(NREL, 16, 16, H).transpose(0, 2, 1, 3).reshape(NREL, 256, H)
    r1b = R1b.reshape(16, 16, H).transpose(1, 0, 2).reshape(256, H)
    w1a = W1a.reshape(NREL, 256, 2, H).transpose(0, 2, 1, 3).reshape(NREL, 512, H)
    r1a = R1a.reshape(256, 2, H).transpose(1, 0, 2).reshape(512, H)
    w1bcat = jnp.concatenate([w1b[0], w1b[1], w1b[2], r1b], axis=1)
    w1acat = jnp.concatenate([w1a[0], w1a[1], w1a[2], r1a], axis=1)
    w2bcat = jnp.concatenate([W2b[0], W2b[1], W2b[2], R2b], axis=1)
    w2acat = jnp.concatenate([W2a[0], W2a[1], W2a[2], R2a], axis=1)
    b1 = jnp.concatenate([b1b, b1a]).reshape(1, HH)
    b2 = jnp.concatenate([b2b, b2a]).reshape(1, HH)
    ang2d = angle_fea.reshape(N, 256)
    spec = jnp.pad(species.astype(jnp.int32), (0, NP - N))
    nbr3 = jnp.pad(nbr_idx.reshape(-1).astype(jnp.int32),
                   (0, EPAD - E)).reshape(16, NCH, 128)
    ones_h = jnp.ones((128, 16), f32)
    zer16 = jnp.zeros((RPT, 16), f32)
    zer64 = jnp.zeros((RPT, H), f32)
    trash_h = jnp.full((EPT,), TRASH, jnp.int32)
    zidx_h = jnp.zeros((EPT,), jnp.int32)

    sc_index, sc_scatter, sc_scatter_cnt = _sc_kernels()
    seg3, gidx3, ecnt = sc_index(spec, nbr3, trash_h, zidx_h)
    seg4 = seg3.reshape(2, 16, NCH, 128)
    gidx4 = gidx3.reshape(2, 16, NCH, 128)
    z1b, z1a, root1 = _tc_a(bond_fea, ang2d, w1bcat, w1acat, b1)
    acc1b, acc1a, cnt = sc_scatter_cnt(
        z1b.reshape(ZROWS, H), z1a.reshape(ZROWS, H),
        seg4, gidx4, zer64, ecnt, ones_h, zer16)
    z2b, z2a, root2 = _tc_b(acc1b.reshape(2, NREL, PLANE, H),
                            acc1a.reshape(2, NREL, PLANE, H),
                            cnt.reshape(2, NREL, PLANE, 16),
                            root1, w2bcat, w2acat, b2)
    acc2b, acc2a = sc_scatter(z2b.reshape(ZROWS, H), z2a.reshape(ZROWS, H),
                              seg4, gidx4, zer64, ecnt)
    return _tc_c(acc2b.reshape(2, NREL, PLANE, H),
                 acc2a.reshape(2, NREL, PLANE, H),
                 cnt.reshape(2, NREL, PLANE, 16),
                 root2, fcW, fcb.reshape(1, 2))


def kernel(bond_fea, angle_fea, species, nbr_idx, crys_idx,
           W1b, R1b, b1b, W1a, R1a, b1a,
           W2b, R2b, b2b, W2a, R2a, b2a, fcW, fcb):
    del crys_idx
    return _run(bond_fea, angle_fea, species, nbr_idx,
                W1b, R1b, b1b, W1a, R1a, b1a,
                W2b, R2b, b2b, W2a, R2a, b2a, fcW, fcb)
